# baseline (speedup 1.0000x reference)
import jax
import jax.numpy as jnp
from jax import lax
from jax.experimental import pallas as pl
from jax.experimental.pallas import tpu as pltpu
from jax.experimental.pallas import tpu_sc as plsc

N = 10000
E = 320000
D_FEAT = 128
D_EDGE = 16
DIM1 = 64
DIM2 = 64
GROUPS = 10
NUM_GRAPHS = 64
GC_COUNT = 3
LAMDA = 0.01
BN_EPS = 1e-5
PRIOR_SIGMA = 0.6931471805599453
LOG_PRIOR = -0.36651292058166435
INV_PR2 = 1.0 / (PRIOR_SIGMA * PRIOR_SIGMA)

HI = lax.Precision.HIGHEST

NW = 32
EPW = E // NW
C = 16
NCHUNK = EPW // C
WIDTH = 128
ZR = 200
NZCHUNK = N // ZR

BN = 1000
BE = 2000


def _dot(a, b, dims):
    return lax.dot_general(a, b, (dims, ((), ())), precision=HI,
                           preferred_element_type=jnp.float32)


def _prep_nodes_k(x_ref, pw_ref, pb_ref, wd_ref, ws_ref, h_ref, nd_ref, ns_ref):
    h = jnp.maximum(_dot(x_ref[...], pw_ref[...], ((1,), (1,))) + pb_ref[...], 0.0)
    h_ref[...] = h
    nd_ref[...] = _dot(h, wd_ref[...], ((1,), (1,)))
    ns_ref[...] = _dot(h, ws_ref[...], ((1,), (1,)))


def _prep_nodes(x, pw, pb, wd, ws):
    return pl.pallas_call(
        _prep_nodes_k,
        grid=(N // BN,),
        in_specs=[
            pl.BlockSpec((BN, D_FEAT), lambda i: (i, 0)),
            pl.BlockSpec((DIM1, D_FEAT), lambda i: (0, 0)),
            pl.BlockSpec((1, DIM1), lambda i: (0, 0)),
            pl.BlockSpec((2 * DIM1, DIM1), lambda i: (0, 0)),
            pl.BlockSpec((2 * DIM1, DIM1), lambda i: (0, 0)),
        ],
        out_specs=[
            pl.BlockSpec((BN, DIM1), lambda i: (i, 0)),
            pl.BlockSpec((BN, 2 * DIM1), lambda i: (i, 0)),
            pl.BlockSpec((BN, 2 * DIM1), lambda i: (i, 0)),
        ],
        out_shape=[
            jax.ShapeDtypeStruct((N, DIM1), jnp.float32),
            jax.ShapeDtypeStruct((N, 2 * DIM1), jnp.float32),
            jax.ShapeDtypeStruct((N, 2 * DIM1), jnp.float32),
        ],
    )(x, pw, pb, wd, ws)


def _edge_proj_k(ea_ref, w_ref, b_ref, r_ref):
    r_ref[...] = _dot(ea_ref[...], w_ref[...], ((1,), (1,))) + b_ref[...]


def _edge_proj(edge_attr, w, b):
    return pl.pallas_call(
        _edge_proj_k,
        grid=(E // BE,),
        in_specs=[
            pl.BlockSpec((BE, D_EDGE), lambda i: (i, 0)),
            pl.BlockSpec((2 * DIM1, D_EDGE), lambda i: (0, 0)),
            pl.BlockSpec((1, 2 * DIM1), lambda i: (0, 0)),
        ],
        out_specs=pl.BlockSpec((BE, 2 * DIM1), lambda i: (i, 0)),
        out_shape=jax.ShapeDtypeStruct((E, 2 * DIM1), jnp.float32),
    )(edge_attr, w, b)


def _gather_sc_body(nd_hbm, ns_hbm, di_hbm, si_hbm, zd_hbm, zs_hbm,
                    di_v, si_v, a_v, b_v):
    cid = lax.axis_index("c")
    sid = lax.axis_index("s")
    wid = cid * 16 + sid
    base0 = wid * EPW

    @pl.loop(0, NCHUNK)
    def _(ci):
        base = base0 + ci * C
        pltpu.sync_copy(di_hbm.at[pl.ds(base, C)], di_v)
        pltpu.sync_copy(si_hbm.at[pl.ds(base, C)], si_v)
        pltpu.sync_copy(nd_hbm.at[di_v], a_v)
        pltpu.sync_copy(ns_hbm.at[si_v], b_v)
        pltpu.sync_copy(a_v, zd_hbm.at[pl.ds(base, C)])
        pltpu.sync_copy(b_v, zs_hbm.at[pl.ds(base, C)])


def _gather_pass(nd, ns, dst, src):
    mesh = plsc.VectorSubcoreMesh(core_axis_name="c", subcore_axis_name="s")
    f = pl.kernel(
        _gather_sc_body,
        out_type=[
            jax.ShapeDtypeStruct((E, 2 * DIM1), jnp.float32),
            jax.ShapeDtypeStruct((E, 2 * DIM1), jnp.float32),
        ],
        mesh=mesh,
        scratch_types=[
            pltpu.VMEM((C,), jnp.int32),
            pltpu.VMEM((C,), jnp.int32),
            pltpu.VMEM((C, 2 * DIM1), jnp.float32),
            pltpu.VMEM((C, 2 * DIM1), jnp.float32),
        ],
    )
    return f(nd, ns, dst, src)


def _gate_k(zd_ref, zs_ref, r_ref, m_ref):
    zf = (zd_ref[:, 0:DIM1] + zs_ref[:, 0:DIM1] + r_ref[:, 0:DIM1])
    zs2 = (zd_ref[:, DIM1:2 * DIM1] + zs_ref[:, DIM1:2 * DIM1]
           + r_ref[:, DIM1:2 * DIM1])
    m = (1.0 / (1.0 + jnp.exp(-zf))) * jnp.logaddexp(zs2, 0.0)
    m_ref[...] = jnp.concatenate(
        [m, jnp.ones((BE, WIDTH - DIM1), jnp.float32)], axis=1)


def _gate(zd, zs, r):
    return pl.pallas_call(
        _gate_k,
        grid=(E // BE,),
        in_specs=[
            pl.BlockSpec((BE, 2 * DIM1), lambda i: (i, 0)),
            pl.BlockSpec((BE, 2 * DIM1), lambda i: (i, 0)),
            pl.BlockSpec((BE, 2 * DIM1), lambda i: (i, 0)),
        ],
        out_specs=pl.BlockSpec((BE, WIDTH), lambda i: (i, 0)),
        out_shape=jax.ShapeDtypeStruct((E, WIDTH), jnp.float32),
    )(zd, zs, r)


def _scatter_sc_body(mw_hbm, di_hbm, out_hbm, di_v, m_v, z_v, s_sh):
    cid = lax.axis_index("c")
    sid = lax.axis_index("s")
    wid = cid * 16 + sid

    zero16 = jnp.zeros((16,), jnp.float32)

    @pl.loop(0, ZR)
    def _(i):
        for k in range(WIDTH // 16):
            z_v[i, pl.ds(k * 16, 16)] = zero16

    @pl.loop(0, 4)
    def _(j):
        ch = sid + j * 16

        @pl.when(ch < NZCHUNK)
        def _():
            pltpu.sync_copy(z_v, s_sh.at[pl.ds(ch * ZR, ZR)])

    plsc.subcore_barrier()

    base0 = wid * EPW

    @pl.loop(0, NCHUNK)
    def _(ci):
        base = base0 + ci * C
        pltpu.sync_copy(di_hbm.at[pl.ds(base, C)], di_v)
        pltpu.sync_copy(mw_hbm.at[pl.ds(base, C)], m_v)
        pltpu.sync_copy(m_v, s_sh.at[di_v], add=True)

    plsc.subcore_barrier()

    @pl.loop(0, 4)
    def _(j):
        ch = sid + j * 16

        @pl.when(ch < NZCHUNK)
        def _():
            pltpu.sync_copy(s_sh.at[pl.ds(ch * ZR, ZR)],
                            out_hbm.at[cid, pl.ds(ch * ZR, ZR)])


def _scatter_pass(mw, dst):
    mesh = plsc.VectorSubcoreMesh(core_axis_name="c", subcore_axis_name="s")
    f = pl.kernel(
        _scatter_sc_body,
        out_type=jax.ShapeDtypeStruct((2, N, WIDTH), jnp.float32),
        mesh=mesh,
        scratch_types=[
            pltpu.VMEM((C,), jnp.int32),
            pltpu.VMEM((C, WIDTH), jnp.float32),
            pltpu.VMEM((ZR, WIDTH), jnp.float32),
            pltpu.VMEM_SHARED((N, WIDTH), jnp.float32),
        ],
    )
    return f(mw, dst)


def _edge_pass(nd, ns, r, dst, src):
    zd, zs = _gather_pass(nd, ns, dst, src)
    mw = _gate(zd, zs, r)
    return _scatter_pass(mw, dst)


def _moments_k(h_ref, s2_ref, gw_ref, gb_ref, h2_ref, sft_ref, mu_ref, ex2_ref):
    i = pl.program_id(0)
    s = s2_ref[0, :, 0:DIM1] + s2_ref[1, :, 0:DIM1]
    cnt = s2_ref[0, :, DIM1:DIM1 + 1] + s2_ref[1, :, DIM1:DIM1 + 1]
    h2 = h_ref[...] + s / jnp.maximum(cnt, 1.0)
    h2_ref[...] = h2
    logits = _dot(h2, gw_ref[...], ((1,), (1,))) + gb_ref[...]
    mx = jnp.max(logits, axis=1, keepdims=True)
    ex = jnp.exp(logits - mx)
    sft = ex / jnp.sum(ex, axis=1, keepdims=True)
    sft_ref[...] = sft
    mu_p = _dot(sft, h2, ((0,), (0,))) * (1.0 / N)
    ex2_p = _dot(sft * sft, h2 * h2, ((0,), (0,))) * (1.0 / N)

    @pl.when(i == 0)
    def _():
        mu_ref[...] = mu_p
        ex2_ref[...] = ex2_p

    @pl.when(i > 0)
    def _():
        mu_ref[...] += mu_p
        ex2_ref[...] += ex2_p


def _moments(h, s2, gw, gb):
    return pl.pallas_call(
        _moments_k,
        grid=(N // BN,),
        in_specs=[
            pl.BlockSpec((BN, DIM1), lambda i: (i, 0)),
            pl.BlockSpec((2, BN, WIDTH), lambda i: (0, i, 0)),
            pl.BlockSpec((GROUPS, DIM1), lambda i: (0, 0)),
            pl.BlockSpec((1, GROUPS), lambda i: (0, 0)),
        ],
        out_specs=[
            pl.BlockSpec((BN, DIM1), lambda i: (i, 0)),
            pl.BlockSpec((BN, GROUPS), lambda i: (i, 0)),
            pl.BlockSpec((GROUPS, DIM1), lambda i: (0, 0)),
            pl.BlockSpec((GROUPS, DIM1), lambda i: (0, 0)),
        ],
        out_shape=[
            jax.ShapeDtypeStruct((N, DIM1), jnp.float32),
            jax.ShapeDtypeStruct((N, GROUPS), jnp.float32),
            jax.ShapeDtypeStruct((GROUPS, DIM1), jnp.float32),
            jax.ShapeDtypeStruct((GROUPS, DIM1), jnp.float32),
        ],
    )(h, s2, gw, gb)


def _dgn_block(h, h2, sft, mu_ref, ex2_ref, gam_ref, bet_ref):
    var = ex2_ref[...] - mu_ref[...] * mu_ref[...]
    a = gam_ref[...] / jnp.sqrt(var + BN_EPS)
    c = jnp.sum(bet_ref[...] - mu_ref[...] * a, axis=0, keepdims=True)
    dgn = h2 + LAMDA * (h2 * _dot(sft, a, ((1,), (0,))) + c)
    return dgn + h


def _apply_k(h_ref, h2_ref, sft_ref, mu_ref, ex2_ref, gam_ref, bet_ref,
             wd_ref, ws_ref, hn_ref, nd_ref, ns_ref):
    hn = _dgn_block(h_ref[...], h2_ref[...], sft_ref[...], mu_ref, ex2_ref,
                    gam_ref, bet_ref)
    hn_ref[...] = hn
    nd_ref[...] = _dot(hn, wd_ref[...], ((1,), (1,)))
    ns_ref[...] = _dot(hn, ws_ref[...], ((1,), (1,)))


def _apply(h, h2, sft, mu, ex2, gam, bet, wd, ws):
    small = lambda shape: pl.BlockSpec(shape, lambda i: (0, 0))
    return pl.pallas_call(
        _apply_k,
        grid=(N // BN,),
        in_specs=[
            pl.BlockSpec((BN, DIM1), lambda i: (i, 0)),
            pl.BlockSpec((BN, DIM1), lambda i: (i, 0)),
            pl.BlockSpec((BN, GROUPS), lambda i: (i, 0)),
            small((GROUPS, DIM1)),
            small((GROUPS, DIM1)),
            small((GROUPS, DIM1)),
            small((GROUPS, DIM1)),
            small((2 * DIM1, DIM1)),
            small((2 * DIM1, DIM1)),
        ],
        out_specs=[
            pl.BlockSpec((BN, DIM1), lambda i: (i, 0)),
            pl.BlockSpec((BN, 2 * DIM1), lambda i: (i, 0)),
            pl.BlockSpec((BN, 2 * DIM1), lambda i: (i, 0)),
        ],
        out_shape=[
            jax.ShapeDtypeStruct((N, DIM1), jnp.float32),
            jax.ShapeDtypeStruct((N, 2 * DIM1), jnp.float32),
            jax.ShapeDtypeStruct((N, 2 * DIM1), jnp.float32),
        ],
    )(h, h2, sft, mu, ex2, gam, bet, wd, ws)


def _pool_k(h_ref, h2_ref, sft_ref, mu_ref, ex2_ref, gam_ref, bet_ref, bt_ref,
            gsum_ref, cnts_ref):
    i = pl.program_id(0)
    hn = _dgn_block(h_ref[...], h2_ref[...], sft_ref[...], mu_ref, ex2_ref,
                    gam_ref, bet_ref)
    cols = lax.broadcasted_iota(jnp.int32, (BN, NUM_GRAPHS), 1)
    oneh = (bt_ref[...] == cols).astype(jnp.float32)
    gs = _dot(oneh, hn, ((0,), (0,)))
    cn = _dot(oneh, jnp.ones((BN, 1), jnp.float32), ((0,), (0,)))

    @pl.when(i == 0)
    def _():
        gsum_ref[...] = gs
        cnts_ref[...] = cn

    @pl.when(i > 0)
    def _():
        gsum_ref[...] += gs
        cnts_ref[...] += cn


def _pool(h, h2, sft, mu, ex2, gam, bet, bt2d):
    small = lambda shape: pl.BlockSpec(shape, lambda i: (0, 0))
    return pl.pallas_call(
        _pool_k,
        grid=(N // BN,),
        in_specs=[
            pl.BlockSpec((BN, DIM1), lambda i: (i, 0)),
            pl.BlockSpec((BN, DIM1), lambda i: (i, 0)),
            pl.BlockSpec((BN, GROUPS), lambda i: (i, 0)),
            small((GROUPS, DIM1)),
            small((GROUPS, DIM1)),
            small((GROUPS, DIM1)),
            small((GROUPS, DIM1)),
            pl.BlockSpec((BN, 1), lambda i: (i, 0)),
        ],
        out_specs=[
            small((NUM_GRAPHS, DIM1)),
            small((NUM_GRAPHS, 1)),
        ],
        out_shape=[
            jax.ShapeDtypeStruct((NUM_GRAPHS, DIM1), jnp.float32),
            jax.ShapeDtypeStruct((NUM_GRAPHS, 1), jnp.float32),
        ],
    )(h, h2, sft, mu, ex2, gam, bet, bt2d)


def _head_k(gsum_ref, cnts_ref,
            pwm_ref, pwr_ref, pbm_ref, pbr_ref,
            owm_ref, owr_ref, obm_ref, obr_ref,
            ew1_ref, eb1_ref, ew2_ref, eb2_ref,
            o_ref, kl_ref):
    g = gsum_ref[...] / jnp.maximum(cnts_ref[...], 1.0)

    def sp(v):
        return jnp.logaddexp(v, 0.0)

    sw1 = sp(pwr_ref[...])
    sb1 = sp(pbr_ref[...])
    w1 = pwm_ref[...] + sw1 * ew1_ref[...]
    b1 = pbm_ref[...] + sb1 * eb1_ref[...]
    h1 = jnp.maximum(_dot(g, w1, ((1,), (1,))) + b1, 0.0)

    sw2 = sp(owr_ref[...])
    sb2 = sp(obr_ref[...])
    w2 = owm_ref[...] + sw2 * ew2_ref[...]
    b2 = obm_ref[...] + sb2 * eb2_ref[...]
    o_ref[...] = jnp.sum(h1 * w2, axis=1, keepdims=True) + b2

    def kld(mu, sig):
        return 0.5 * jnp.sum(2.0 * (LOG_PRIOR - jnp.log(sig))
                             + (sig * sig + mu * mu) * INV_PR2 - 1.0)

    kl = (kld(pwm_ref[...], sw1) + kld(pbm_ref[...], sb1)
          + kld(owm_ref[...], sw2) + kld(obm_ref[...], sb2))
    kl_ref[...] = jnp.reshape(kl, (1, 1))


def _head(gsum, cnts, pwm, pwr, pbm, pbr, owm, owr, obm, obr,
          ew1, eb1, ew2, eb2):
    return pl.pallas_call(
        _head_k,
        out_shape=[
            jax.ShapeDtypeStruct((NUM_GRAPHS, 1), jnp.float32),
            jax.ShapeDtypeStruct((1, 1), jnp.float32),
        ],
    )(gsum, cnts, pwm, pwr, pbm, pbr, owm, owr, obm, obr,
      ew1, eb1, ew2, eb2)


def kernel(x, edge_index, edge_attr, batch, params):
    src = edge_index[0]
    dst = edge_index[1]
    convs = params['convs']

    wd = [jnp.concatenate([c['Wf'][:, 0:DIM1], c['Ws'][:, 0:DIM1]], axis=0)
          for c in convs]
    wsr = [jnp.concatenate([c['Wf'][:, DIM1:2 * DIM1], c['Ws'][:, DIM1:2 * DIM1]],
                           axis=0) for c in convs]
    we = [jnp.concatenate([c['Wf'][:, 2 * DIM1:], c['Ws'][:, 2 * DIM1:]], axis=0)
          for c in convs]
    be = [jnp.concatenate([c['bf'], c['bs']]).reshape(1, -1) for c in convs]

    h, nd, ns = _prep_nodes(x, params['pre_W'], params['pre_b'].reshape(1, -1),
                            wd[0], wsr[0])
    rs = [_edge_proj(edge_attr, we[l], be[l]) for l in range(GC_COUNT)]

    kb1, kb2 = jax.random.split(jax.random.key(42))
    k11, k12 = jax.random.split(kb1)
    k21, k22 = jax.random.split(kb2)
    ew1 = jax.random.normal(k11, (DIM2, DIM1), dtype=jnp.float32)
    eb1 = jax.random.normal(k12, (DIM2,), dtype=jnp.float32).reshape(1, -1)
    ew2 = jax.random.normal(k21, (1, DIM2), dtype=jnp.float32)
    eb2 = jax.random.normal(k22, (1,), dtype=jnp.float32).reshape(1, 1)

    for l in range(GC_COUNT):
        c = convs[l]
        s2 = _edge_pass(nd, ns, rs[l], dst, src)
        gb = c['gn_b'].reshape(1, -1)
        h2, sft, mu, ex2 = _moments(h, s2, c['gn_W'], gb)
        if l < GC_COUNT - 1:
            h, nd, ns = _apply(h, h2, sft, mu, ex2, c['gamma'], c['beta'],
                               wd[l + 1], wsr[l + 1])
        else:
            gsum, cnts = _pool(h, h2, sft, mu, ex2, c['gamma'], c['beta'],
                               batch.reshape(-1, 1))
            o, kl = _head(gsum, cnts,
                          params['post_Wmu'], params['post_Wrho'],
                          params['post_bmu'].reshape(1, -1),
                          params['post_brho'].reshape(1, -1),
                          params['out_Wmu'], params['out_Wrho'],
                          params['out_bmu'].reshape(1, 1),
                          params['out_brho'].reshape(1, 1),
                          ew1, eb1, ew2, eb2)

    return o.reshape(-1), kl.reshape(())


# speedup vs baseline: 2.2762x; 2.2762x over previous
import jax
import jax.numpy as jnp
from jax import lax
from jax.experimental import pallas as pl
from jax.experimental.pallas import tpu as pltpu
from jax.experimental.pallas import tpu_sc as plsc

N = 10000
E = 320000
D_FEAT = 128
D_EDGE = 16
DIM1 = 64
DIM2 = 64
GROUPS = 10
NUM_GRAPHS = 64
GC_COUNT = 3
LAMDA = 0.01
BN_EPS = 1e-5
PRIOR_SIGMA = 0.6931471805599453
LOG_PRIOR = -0.36651292058166435
INV_PR2 = 1.0 / (PRIOR_SIGMA * PRIOR_SIGMA)

HI = lax.Precision.HIGHEST

NW = 32
EPW = E // NW
C = 40
NCHUNK = EPW // C
WIDTH = 128
ZR = 200
NZCHUNK = N // ZR

BN = 1000
BE = 2000


def _dot(a, b, dims):
    return lax.dot_general(a, b, (dims, ((), ())), precision=HI,
                           preferred_element_type=jnp.float32)


def _prep_nodes_k(x_ref, pw_ref, pb_ref, wd_ref, ws_ref, h_ref, nd_ref, ns_ref):
    h = jnp.maximum(_dot(x_ref[...], pw_ref[...], ((1,), (1,))) + pb_ref[...], 0.0)
    h_ref[...] = h
    nd_ref[...] = _dot(h, wd_ref[...], ((1,), (1,)))
    ns_ref[...] = _dot(h, ws_ref[...], ((1,), (1,)))


def _prep_nodes(x, pw, pb, wd, ws):
    return pl.pallas_call(
        _prep_nodes_k,
        grid=(N // BN,),
        in_specs=[
            pl.BlockSpec((BN, D_FEAT), lambda i: (i, 0)),
            pl.BlockSpec((DIM1, D_FEAT), lambda i: (0, 0)),
            pl.BlockSpec((1, DIM1), lambda i: (0, 0)),
            pl.BlockSpec((2 * DIM1, DIM1), lambda i: (0, 0)),
            pl.BlockSpec((2 * DIM1, DIM1), lambda i: (0, 0)),
        ],
        out_specs=[
            pl.BlockSpec((BN, DIM1), lambda i: (i, 0)),
            pl.BlockSpec((BN, 2 * DIM1), lambda i: (i, 0)),
            pl.BlockSpec((BN, 2 * DIM1), lambda i: (i, 0)),
        ],
        out_shape=[
            jax.ShapeDtypeStruct((N, DIM1), jnp.float32),
            jax.ShapeDtypeStruct((N, 2 * DIM1), jnp.float32),
            jax.ShapeDtypeStruct((N, 2 * DIM1), jnp.float32),
        ],
    )(x, pw, pb, wd, ws)


def _edge_proj_k(ea_ref, w_ref, b_ref, r_ref):
    r_ref[...] = _dot(ea_ref[...], w_ref[...], ((1,), (1,))) + b_ref[...]


def _edge_proj(edge_attr, w, b):
    return pl.pallas_call(
        _edge_proj_k,
        grid=(E // BE,),
        in_specs=[
            pl.BlockSpec((BE, D_EDGE), lambda i: (i, 0)),
            pl.BlockSpec((2 * DIM1, D_EDGE), lambda i: (0, 0)),
            pl.BlockSpec((1, 2 * DIM1), lambda i: (0, 0)),
        ],
        out_specs=pl.BlockSpec((BE, 2 * DIM1), lambda i: (i, 0)),
        out_shape=jax.ShapeDtypeStruct((E, 2 * DIM1), jnp.float32),
    )(edge_attr, w, b)


def _gather_sc_body(nd_hbm, ns_hbm, di_hbm, si_hbm, zd_hbm, zs_hbm,
                    di_v, si_v, a_v, b_v, sem1, sem2):
    cid = lax.axis_index("c")
    sid = lax.axis_index("s")
    wid = cid * 16 + sid
    base0 = wid * EPW

    @pl.loop(0, NCHUNK)
    def _(ci):
        base = base0 + ci * C
        c1 = pltpu.async_copy(di_hbm.at[pl.ds(base, C)], di_v, sem1)
        c2 = pltpu.async_copy(si_hbm.at[pl.ds(base, C)], si_v, sem2)
        c1.wait()
        c2.wait()
        c3 = pltpu.async_copy(nd_hbm.at[di_v], a_v, sem1)
        c4 = pltpu.async_copy(ns_hbm.at[si_v], b_v, sem2)
        c3.wait()
        c4.wait()
        c5 = pltpu.async_copy(a_v, zd_hbm.at[pl.ds(base, C)], sem1)
        c6 = pltpu.async_copy(b_v, zs_hbm.at[pl.ds(base, C)], sem2)
        c5.wait()
        c6.wait()


def _gather_pass(nd, ns, dst, src):
    mesh = plsc.VectorSubcoreMesh(core_axis_name="c", subcore_axis_name="s")
    f = pl.kernel(
        _gather_sc_body,
        out_type=[
            jax.ShapeDtypeStruct((E, 2 * DIM1), jnp.float32),
            jax.ShapeDtypeStruct((E, 2 * DIM1), jnp.float32),
        ],
        mesh=mesh,
        scratch_types=[
            pltpu.VMEM((C,), jnp.int32),
            pltpu.VMEM((C,), jnp.int32),
            pltpu.VMEM((C, 2 * DIM1), jnp.float32),
            pltpu.VMEM((C, 2 * DIM1), jnp.float32),
            pltpu.SemaphoreType.DMA,
            pltpu.SemaphoreType.DMA,
        ],
    )
    return f(nd, ns, dst, src)


def _gate_k(zd_ref, zs_ref, r_ref, m_ref):
    zf = (zd_ref[:, 0:DIM1] + zs_ref[:, 0:DIM1] + r_ref[:, 0:DIM1])
    zs2 = (zd_ref[:, DIM1:2 * DIM1] + zs_ref[:, DIM1:2 * DIM1]
           + r_ref[:, DIM1:2 * DIM1])
    m = (1.0 / (1.0 + jnp.exp(-zf))) * jnp.logaddexp(zs2, 0.0)
    m_ref[...] = jnp.concatenate(
        [m, jnp.ones((BE, WIDTH - DIM1), jnp.float32)], axis=1)


def _gate(zd, zs, r):
    return pl.pallas_call(
        _gate_k,
        grid=(E // BE,),
        in_specs=[
            pl.BlockSpec((BE, 2 * DIM1), lambda i: (i, 0)),
            pl.BlockSpec((BE, 2 * DIM1), lambda i: (i, 0)),
            pl.BlockSpec((BE, 2 * DIM1), lambda i: (i, 0)),
        ],
        out_specs=pl.BlockSpec((BE, WIDTH), lambda i: (i, 0)),
        out_shape=jax.ShapeDtypeStruct((E, WIDTH), jnp.float32),
    )(zd, zs, r)


def _scatter_sc_body(mw_hbm, di_hbm, out_hbm, di_v, m_v, z_v, s_sh, sem1, sem2):
    cid = lax.axis_index("c")
    sid = lax.axis_index("s")
    wid = cid * 16 + sid

    zero16 = jnp.zeros((16,), jnp.float32)

    @pl.loop(0, ZR)
    def _(i):
        for k in range(WIDTH // 16):
            z_v[i, pl.ds(k * 16, 16)] = zero16

    @pl.loop(0, 4)
    def _(j):
        ch = sid + j * 16

        @pl.when(ch < NZCHUNK)
        def _():
            pltpu.sync_copy(z_v, s_sh.at[pl.ds(ch * ZR, ZR)])

    plsc.subcore_barrier()

    base0 = wid * EPW

    @pl.loop(0, NCHUNK)
    def _(ci):
        base = base0 + ci * C
        c1 = pltpu.async_copy(di_hbm.at[pl.ds(base, C)], di_v, sem1)
        c2 = pltpu.async_copy(mw_hbm.at[pl.ds(base, C)], m_v, sem2)
        c1.wait()
        c2.wait()
        pltpu.sync_copy(m_v, s_sh.at[di_v], add=True)

    plsc.subcore_barrier()

    @pl.loop(0, 4)
    def _(j):
        ch = sid + j * 16

        @pl.when(ch < NZCHUNK)
        def _():
            pltpu.sync_copy(s_sh.at[pl.ds(ch * ZR, ZR)],
                            out_hbm.at[cid, pl.ds(ch * ZR, ZR)])


def _scatter_pass(mw, dst):
    mesh = plsc.VectorSubcoreMesh(core_axis_name="c", subcore_axis_name="s")
    f = pl.kernel(
        _scatter_sc_body,
        out_type=jax.ShapeDtypeStruct((2, N, WIDTH), jnp.float32),
        mesh=mesh,
        scratch_types=[
            pltpu.VMEM((C,), jnp.int32),
            pltpu.VMEM((C, WIDTH), jnp.float32),
            pltpu.VMEM((ZR, WIDTH), jnp.float32),
            pltpu.VMEM_SHARED((N, WIDTH), jnp.float32),
            pltpu.SemaphoreType.DMA,
            pltpu.SemaphoreType.DMA,
        ],
    )
    return f(mw, dst)


def _edge_pass(nd, ns, r, dst, src):
    zd, zs = _gather_pass(nd, ns, dst, src)
    mw = _gate(zd, zs, r)
    return _scatter_pass(mw, dst)


def _moments_k(h_ref, s2_ref, gw_ref, gb_ref, h2_ref, sft_ref, mu_ref, ex2_ref):
    i = pl.program_id(0)
    s = s2_ref[0, :, 0:DIM1] + s2_ref[1, :, 0:DIM1]
    cnt = s2_ref[0, :, DIM1:DIM1 + 1] + s2_ref[1, :, DIM1:DIM1 + 1]
    h2 = h_ref[...] + s / jnp.maximum(cnt, 1.0)
    h2_ref[...] = h2
    logits = _dot(h2, gw_ref[...], ((1,), (1,))) + gb_ref[...]
    mx = jnp.max(logits, axis=1, keepdims=True)
    ex = jnp.exp(logits - mx)
    sft = ex / jnp.sum(ex, axis=1, keepdims=True)
    sft_ref[...] = sft
    mu_p = _dot(sft, h2, ((0,), (0,))) * (1.0 / N)
    ex2_p = _dot(sft * sft, h2 * h2, ((0,), (0,))) * (1.0 / N)

    @pl.when(i == 0)
    def _():
        mu_ref[...] = mu_p
        ex2_ref[...] = ex2_p

    @pl.when(i > 0)
    def _():
        mu_ref[...] += mu_p
        ex2_ref[...] += ex2_p


def _moments(h, s2, gw, gb):
    return pl.pallas_call(
        _moments_k,
        grid=(N // BN,),
        in_specs=[
            pl.BlockSpec((BN, DIM1), lambda i: (i, 0)),
            pl.BlockSpec((2, BN, WIDTH), lambda i: (0, i, 0)),
            pl.BlockSpec((GROUPS, DIM1), lambda i: (0, 0)),
            pl.BlockSpec((1, GROUPS), lambda i: (0, 0)),
        ],
        out_specs=[
            pl.BlockSpec((BN, DIM1), lambda i: (i, 0)),
            pl.BlockSpec((BN, GROUPS), lambda i: (i, 0)),
            pl.BlockSpec((GROUPS, DIM1), lambda i: (0, 0)),
            pl.BlockSpec((GROUPS, DIM1), lambda i: (0, 0)),
        ],
        out_shape=[
            jax.ShapeDtypeStruct((N, DIM1), jnp.float32),
            jax.ShapeDtypeStruct((N, GROUPS), jnp.float32),
            jax.ShapeDtypeStruct((GROUPS, DIM1), jnp.float32),
            jax.ShapeDtypeStruct((GROUPS, DIM1), jnp.float32),
        ],
    )(h, s2, gw, gb)


def _dgn_block(h, h2, sft, mu_ref, ex2_ref, gam_ref, bet_ref):
    var = ex2_ref[...] - mu_ref[...] * mu_ref[...]
    a = gam_ref[...] / jnp.sqrt(var + BN_EPS)
    c = jnp.sum(bet_ref[...] - mu_ref[...] * a, axis=0, keepdims=True)
    dgn = h2 + LAMDA * (h2 * _dot(sft, a, ((1,), (0,))) + c)
    return dgn + h


def _apply_k(h_ref, h2_ref, sft_ref, mu_ref, ex2_ref, gam_ref, bet_ref,
             wd_ref, ws_ref, hn_ref, nd_ref, ns_ref):
    hn = _dgn_block(h_ref[...], h2_ref[...], sft_ref[...], mu_ref, ex2_ref,
                    gam_ref, bet_ref)
    hn_ref[...] = hn
    nd_ref[...] = _dot(hn, wd_ref[...], ((1,), (1,)))
    ns_ref[...] = _dot(hn, ws_ref[...], ((1,), (1,)))


def _apply(h, h2, sft, mu, ex2, gam, bet, wd, ws):
    small = lambda shape: pl.BlockSpec(shape, lambda i: (0, 0))
    return pl.pallas_call(
        _apply_k,
        grid=(N // BN,),
        in_specs=[
            pl.BlockSpec((BN, DIM1), lambda i: (i, 0)),
            pl.BlockSpec((BN, DIM1), lambda i: (i, 0)),
            pl.BlockSpec((BN, GROUPS), lambda i: (i, 0)),
            small((GROUPS, DIM1)),
            small((GROUPS, DIM1)),
            small((GROUPS, DIM1)),
            small((GROUPS, DIM1)),
            small((2 * DIM1, DIM1)),
            small((2 * DIM1, DIM1)),
        ],
        out_specs=[
            pl.BlockSpec((BN, DIM1), lambda i: (i, 0)),
            pl.BlockSpec((BN, 2 * DIM1), lambda i: (i, 0)),
            pl.BlockSpec((BN, 2 * DIM1), lambda i: (i, 0)),
        ],
        out_shape=[
            jax.ShapeDtypeStruct((N, DIM1), jnp.float32),
            jax.ShapeDtypeStruct((N, 2 * DIM1), jnp.float32),
            jax.ShapeDtypeStruct((N, 2 * DIM1), jnp.float32),
        ],
    )(h, h2, sft, mu, ex2, gam, bet, wd, ws)


def _pool_k(h_ref, h2_ref, sft_ref, mu_ref, ex2_ref, gam_ref, bet_ref, bt_ref,
            gsum_ref, cnts_ref):
    i = pl.program_id(0)
    hn = _dgn_block(h_ref[...], h2_ref[...], sft_ref[...], mu_ref, ex2_ref,
                    gam_ref, bet_ref)
    cols = lax.broadcasted_iota(jnp.int32, (BN, NUM_GRAPHS), 1)
    oneh = (bt_ref[...] == cols).astype(jnp.float32)
    gs = _dot(oneh, hn, ((0,), (0,)))
    cn = _dot(oneh, jnp.ones((BN, 1), jnp.float32), ((0,), (0,)))

    @pl.when(i == 0)
    def _():
        gsum_ref[...] = gs
        cnts_ref[...] = cn

    @pl.when(i > 0)
    def _():
        gsum_ref[...] += gs
        cnts_ref[...] += cn


def _pool(h, h2, sft, mu, ex2, gam, bet, bt2d):
    small = lambda shape: pl.BlockSpec(shape, lambda i: (0, 0))
    return pl.pallas_call(
        _pool_k,
        grid=(N // BN,),
        in_specs=[
            pl.BlockSpec((BN, DIM1), lambda i: (i, 0)),
            pl.BlockSpec((BN, DIM1), lambda i: (i, 0)),
            pl.BlockSpec((BN, GROUPS), lambda i: (i, 0)),
            small((GROUPS, DIM1)),
            small((GROUPS, DIM1)),
            small((GROUPS, DIM1)),
            small((GROUPS, DIM1)),
            pl.BlockSpec((BN, 1), lambda i: (i, 0)),
        ],
        out_specs=[
            small((NUM_GRAPHS, DIM1)),
            small((NUM_GRAPHS, 1)),
        ],
        out_shape=[
            jax.ShapeDtypeStruct((NUM_GRAPHS, DIM1), jnp.float32),
            jax.ShapeDtypeStruct((NUM_GRAPHS, 1), jnp.float32),
        ],
    )(h, h2, sft, mu, ex2, gam, bet, bt2d)


def _head_k(gsum_ref, cnts_ref,
            pwm_ref, pwr_ref, pbm_ref, pbr_ref,
            owm_ref, owr_ref, obm_ref, obr_ref,
            ew1_ref, eb1_ref, ew2_ref, eb2_ref,
            o_ref, kl_ref):
    g = gsum_ref[...] / jnp.maximum(cnts_ref[...], 1.0)

    def sp(v):
        return jnp.logaddexp(v, 0.0)

    sw1 = sp(pwr_ref[...])
    sb1 = sp(pbr_ref[...])
    w1 = pwm_ref[...] + sw1 * ew1_ref[...]
    b1 = pbm_ref[...] + sb1 * eb1_ref[...]
    h1 = jnp.maximum(_dot(g, w1, ((1,), (1,))) + b1, 0.0)

    sw2 = sp(owr_ref[...])
    sb2 = sp(obr_ref[...])
    w2 = owm_ref[...] + sw2 * ew2_ref[...]
    b2 = obm_ref[...] + sb2 * eb2_ref[...]
    o_ref[...] = jnp.sum(h1 * w2, axis=1, keepdims=True) + b2

    def kld(mu, sig):
        return 0.5 * jnp.sum(2.0 * (LOG_PRIOR - jnp.log(sig))
                             + (sig * sig + mu * mu) * INV_PR2 - 1.0)

    kl = (kld(pwm_ref[...], sw1) + kld(pbm_ref[...], sb1)
          + kld(owm_ref[...], sw2) + kld(obm_ref[...], sb2))
    kl_ref[...] = jnp.reshape(kl, (1, 1))


def _head(gsum, cnts, pwm, pwr, pbm, pbr, owm, owr, obm, obr,
          ew1, eb1, ew2, eb2):
    return pl.pallas_call(
        _head_k,
        out_shape=[
            jax.ShapeDtypeStruct((NUM_GRAPHS, 1), jnp.float32),
            jax.ShapeDtypeStruct((1, 1), jnp.float32),
        ],
    )(gsum, cnts, pwm, pwr, pbm, pbr, owm, owr, obm, obr,
      ew1, eb1, ew2, eb2)


def kernel(x, edge_index, edge_attr, batch, params):
    src = edge_index[0]
    dst = edge_index[1]
    convs = params['convs']

    wd = [jnp.concatenate([c['Wf'][:, 0:DIM1], c['Ws'][:, 0:DIM1]], axis=0)
          for c in convs]
    wsr = [jnp.concatenate([c['Wf'][:, DIM1:2 * DIM1], c['Ws'][:, DIM1:2 * DIM1]],
                           axis=0) for c in convs]
    we = [jnp.concatenate([c['Wf'][:, 2 * DIM1:], c['Ws'][:, 2 * DIM1:]], axis=0)
          for c in convs]
    be = [jnp.concatenate([c['bf'], c['bs']]).reshape(1, -1) for c in convs]

    h, nd, ns = _prep_nodes(x, params['pre_W'], params['pre_b'].reshape(1, -1),
                            wd[0], wsr[0])
    rs = [_edge_proj(edge_attr, we[l], be[l]) for l in range(GC_COUNT)]

    kb1, kb2 = jax.random.split(jax.random.key(42))
    k11, k12 = jax.random.split(kb1)
    k21, k22 = jax.random.split(kb2)
    ew1 = jax.random.normal(k11, (DIM2, DIM1), dtype=jnp.float32)
    eb1 = jax.random.normal(k12, (DIM2,), dtype=jnp.float32).reshape(1, -1)
    ew2 = jax.random.normal(k21, (1, DIM2), dtype=jnp.float32)
    eb2 = jax.random.normal(k22, (1,), dtype=jnp.float32).reshape(1, 1)

    for l in range(GC_COUNT):
        c = convs[l]
        s2 = _edge_pass(nd, ns, rs[l], dst, src)
        gb = c['gn_b'].reshape(1, -1)
        h2, sft, mu, ex2 = _moments(h, s2, c['gn_W'], gb)
        if l < GC_COUNT - 1:
            h, nd, ns = _apply(h, h2, sft, mu, ex2, c['gamma'], c['beta'],
                               wd[l + 1], wsr[l + 1])
        else:
            gsum, cnts = _pool(h, h2, sft, mu, ex2, c['gamma'], c['beta'],
                               batch.reshape(-1, 1))
            o, kl = _head(gsum, cnts,
                          params['post_Wmu'], params['post_Wrho'],
                          params['post_bmu'].reshape(1, -1),
                          params['post_brho'].reshape(1, -1),
                          params['out_Wmu'], params['out_Wrho'],
                          params['out_bmu'].reshape(1, 1),
                          params['out_brho'].reshape(1, 1),
                          ew1, eb1, ew2, eb2)

    return o.reshape(-1), kl.reshape(())


# speedup vs baseline: 2.8085x; 1.2338x over previous
import jax
import jax.numpy as jnp
from jax import lax
from jax.experimental import pallas as pl
from jax.experimental.pallas import tpu as pltpu
from jax.experimental.pallas import tpu_sc as plsc

N = 10000
E = 320000
D_FEAT = 128
D_EDGE = 16
DIM1 = 64
DIM2 = 64
GROUPS = 10
NUM_GRAPHS = 64
GC_COUNT = 3
LAMDA = 0.01
BN_EPS = 1e-5
PRIOR_SIGMA = 0.6931471805599453
LOG_PRIOR = -0.36651292058166435
INV_PR2 = 1.0 / (PRIOR_SIGMA * PRIOR_SIGMA)

HI = lax.Precision.HIGHEST

NW = 32
EPW = E // NW
C = 80
NCHUNK = EPW // C
WIDTH = 128
ZR = 200
NZCHUNK = N // ZR

BN = 1000
BE = 2000


def _dot(a, b, dims):
    return lax.dot_general(a, b, (dims, ((), ())), precision=HI,
                           preferred_element_type=jnp.float32)


def _prep_nodes_k(x_ref, pw_ref, pb_ref, wd_ref, ws_ref, h_ref, nd_ref, ns_ref):
    h = jnp.maximum(_dot(x_ref[...], pw_ref[...], ((1,), (1,))) + pb_ref[...], 0.0)
    h_ref[...] = h
    nd_ref[...] = _dot(h, wd_ref[...], ((1,), (1,)))
    ns_ref[...] = _dot(h, ws_ref[...], ((1,), (1,)))


def _prep_nodes(x, pw, pb, wd, ws):
    return pl.pallas_call(
        _prep_nodes_k,
        grid=(N // BN,),
        in_specs=[
            pl.BlockSpec((BN, D_FEAT), lambda i: (i, 0)),
            pl.BlockSpec((DIM1, D_FEAT), lambda i: (0, 0)),
            pl.BlockSpec((1, DIM1), lambda i: (0, 0)),
            pl.BlockSpec((2 * DIM1, DIM1), lambda i: (0, 0)),
            pl.BlockSpec((2 * DIM1, DIM1), lambda i: (0, 0)),
        ],
        out_specs=[
            pl.BlockSpec((BN, DIM1), lambda i: (i, 0)),
            pl.BlockSpec((BN, 2 * DIM1), lambda i: (i, 0)),
            pl.BlockSpec((BN, 2 * DIM1), lambda i: (i, 0)),
        ],
        out_shape=[
            jax.ShapeDtypeStruct((N, DIM1), jnp.float32),
            jax.ShapeDtypeStruct((N, 2 * DIM1), jnp.float32),
            jax.ShapeDtypeStruct((N, 2 * DIM1), jnp.float32),
        ],
    )(x, pw, pb, wd, ws)


def _edge_proj_k(ea_ref, w_ref, b_ref, r_ref):
    r_ref[...] = _dot(ea_ref[...], w_ref[...], ((1,), (1,))) + b_ref[...]


def _edge_proj(edge_attr, w, b):
    return pl.pallas_call(
        _edge_proj_k,
        grid=(E // BE,),
        in_specs=[
            pl.BlockSpec((BE, D_EDGE), lambda i: (i, 0)),
            pl.BlockSpec((2 * DIM1, D_EDGE), lambda i: (0, 0)),
            pl.BlockSpec((1, 2 * DIM1), lambda i: (0, 0)),
        ],
        out_specs=pl.BlockSpec((BE, 2 * DIM1), lambda i: (i, 0)),
        out_shape=jax.ShapeDtypeStruct((E, 2 * DIM1), jnp.float32),
    )(edge_attr, w, b)


def _gather_sc_body(nd_hbm, ns_hbm, di_hbm, si_hbm, zd_hbm, zs_hbm,
                    di_v, si_v, a_v, b_v, sem1, sem2):
    cid = lax.axis_index("c")
    sid = lax.axis_index("s")
    wid = cid * 16 + sid
    base0 = wid * EPW

    @pl.loop(0, NCHUNK)
    def _(ci):
        base = base0 + ci * C
        c1 = pltpu.async_copy(di_hbm.at[pl.ds(base, C)], di_v, sem1)
        c2 = pltpu.async_copy(si_hbm.at[pl.ds(base, C)], si_v, sem2)
        c1.wait()
        c2.wait()
        c3 = pltpu.async_copy(nd_hbm.at[di_v], a_v, sem1)
        c4 = pltpu.async_copy(ns_hbm.at[si_v], b_v, sem2)
        c3.wait()
        c4.wait()
        c5 = pltpu.async_copy(a_v, zd_hbm.at[pl.ds(base, C)], sem1)
        c6 = pltpu.async_copy(b_v, zs_hbm.at[pl.ds(base, C)], sem2)
        c5.wait()
        c6.wait()


def _gather_pass(nd, ns, dst, src):
    mesh = plsc.VectorSubcoreMesh(core_axis_name="c", subcore_axis_name="s")
    f = pl.kernel(
        _gather_sc_body,
        out_type=[
            jax.ShapeDtypeStruct((E, 2 * DIM1), jnp.float32),
            jax.ShapeDtypeStruct((E, 2 * DIM1), jnp.float32),
        ],
        mesh=mesh,
        scratch_types=[
            pltpu.VMEM((C,), jnp.int32),
            pltpu.VMEM((C,), jnp.int32),
            pltpu.VMEM((C, 2 * DIM1), jnp.float32),
            pltpu.VMEM((C, 2 * DIM1), jnp.float32),
            pltpu.SemaphoreType.DMA,
            pltpu.SemaphoreType.DMA,
        ],
    )
    return f(nd, ns, dst, src)


def _gate_k(zd_ref, zs_ref, r_ref, m_ref):
    zf = (zd_ref[:, 0:DIM1] + zs_ref[:, 0:DIM1] + r_ref[:, 0:DIM1])
    zs2 = (zd_ref[:, DIM1:2 * DIM1] + zs_ref[:, DIM1:2 * DIM1]
           + r_ref[:, DIM1:2 * DIM1])
    m = (1.0 / (1.0 + jnp.exp(-zf))) * jnp.logaddexp(zs2, 0.0)
    m_ref[...] = jnp.concatenate(
        [m, jnp.ones((BE, WIDTH - DIM1), jnp.float32)], axis=1)


def _gate(zd, zs, r):
    return pl.pallas_call(
        _gate_k,
        grid=(E // BE,),
        in_specs=[
            pl.BlockSpec((BE, 2 * DIM1), lambda i: (i, 0)),
            pl.BlockSpec((BE, 2 * DIM1), lambda i: (i, 0)),
            pl.BlockSpec((BE, 2 * DIM1), lambda i: (i, 0)),
        ],
        out_specs=pl.BlockSpec((BE, WIDTH), lambda i: (i, 0)),
        out_shape=jax.ShapeDtypeStruct((E, WIDTH), jnp.float32),
    )(zd, zs, r)


def _scatter_sc_body(mw_hbm, di_hbm, out_hbm, di_v, m_v, z_v, s_sh, sem1, sem2):
    cid = lax.axis_index("c")
    sid = lax.axis_index("s")
    wid = cid * 16 + sid

    zero16 = jnp.zeros((16,), jnp.float32)

    @pl.loop(0, ZR)
    def _(i):
        for k in range(WIDTH // 16):
            z_v[i, pl.ds(k * 16, 16)] = zero16

    @pl.loop(0, 4)
    def _(j):
        ch = sid + j * 16

        @pl.when(ch < NZCHUNK)
        def _():
            pltpu.sync_copy(z_v, s_sh.at[pl.ds(ch * ZR, ZR)])

    plsc.subcore_barrier()

    base0 = wid * EPW

    @pl.loop(0, NCHUNK)
    def _(ci):
        base = base0 + ci * C
        c1 = pltpu.async_copy(di_hbm.at[pl.ds(base, C)], di_v, sem1)
        c2 = pltpu.async_copy(mw_hbm.at[pl.ds(base, C)], m_v, sem2)
        c1.wait()
        c2.wait()
        pltpu.sync_copy(m_v, s_sh.at[di_v], add=True)

    plsc.subcore_barrier()

    @pl.loop(0, 4)
    def _(j):
        ch = sid + j * 16

        @pl.when(ch < NZCHUNK)
        def _():
            pltpu.sync_copy(s_sh.at[pl.ds(ch * ZR, ZR)],
                            out_hbm.at[cid, pl.ds(ch * ZR, ZR)])


def _scatter_pass(mw, dst):
    mesh = plsc.VectorSubcoreMesh(core_axis_name="c", subcore_axis_name="s")
    f = pl.kernel(
        _scatter_sc_body,
        out_type=jax.ShapeDtypeStruct((2, N, WIDTH), jnp.float32),
        mesh=mesh,
        scratch_types=[
            pltpu.VMEM((C,), jnp.int32),
            pltpu.VMEM((C, WIDTH), jnp.float32),
            pltpu.VMEM((ZR, WIDTH), jnp.float32),
            pltpu.VMEM_SHARED((N, WIDTH), jnp.float32),
            pltpu.SemaphoreType.DMA,
            pltpu.SemaphoreType.DMA,
        ],
    )
    return f(mw, dst)


def _edge_pass(nd, ns, r, dst, src):
    zd, zs = _gather_pass(nd, ns, dst, src)
    mw = _gate(zd, zs, r)
    return _scatter_pass(mw, dst)


def _moments_k(h_ref, s2_ref, gw_ref, gb_ref, h2_ref, sft_ref, mu_ref, ex2_ref):
    i = pl.program_id(0)
    s = s2_ref[0, :, 0:DIM1] + s2_ref[1, :, 0:DIM1]
    cnt = s2_ref[0, :, DIM1:DIM1 + 1] + s2_ref[1, :, DIM1:DIM1 + 1]
    h2 = h_ref[...] + s / jnp.maximum(cnt, 1.0)
    h2_ref[...] = h2
    logits = _dot(h2, gw_ref[...], ((1,), (1,))) + gb_ref[...]
    mx = jnp.max(logits, axis=1, keepdims=True)
    ex = jnp.exp(logits - mx)
    sft = ex / jnp.sum(ex, axis=1, keepdims=True)
    sft_ref[...] = sft
    mu_p = _dot(sft, h2, ((0,), (0,))) * (1.0 / N)
    ex2_p = _dot(sft * sft, h2 * h2, ((0,), (0,))) * (1.0 / N)

    @pl.when(i == 0)
    def _():
        mu_ref[...] = mu_p
        ex2_ref[...] = ex2_p

    @pl.when(i > 0)
    def _():
        mu_ref[...] += mu_p
        ex2_ref[...] += ex2_p


def _moments(h, s2, gw, gb):
    return pl.pallas_call(
        _moments_k,
        grid=(N // BN,),
        in_specs=[
            pl.BlockSpec((BN, DIM1), lambda i: (i, 0)),
            pl.BlockSpec((2, BN, WIDTH), lambda i: (0, i, 0)),
            pl.BlockSpec((GROUPS, DIM1), lambda i: (0, 0)),
            pl.BlockSpec((1, GROUPS), lambda i: (0, 0)),
        ],
        out_specs=[
            pl.BlockSpec((BN, DIM1), lambda i: (i, 0)),
            pl.BlockSpec((BN, GROUPS), lambda i: (i, 0)),
            pl.BlockSpec((GROUPS, DIM1), lambda i: (0, 0)),
            pl.BlockSpec((GROUPS, DIM1), lambda i: (0, 0)),
        ],
        out_shape=[
            jax.ShapeDtypeStruct((N, DIM1), jnp.float32),
            jax.ShapeDtypeStruct((N, GROUPS), jnp.float32),
            jax.ShapeDtypeStruct((GROUPS, DIM1), jnp.float32),
            jax.ShapeDtypeStruct((GROUPS, DIM1), jnp.float32),
        ],
    )(h, s2, gw, gb)


def _dgn_block(h, h2, sft, mu_ref, ex2_ref, gam_ref, bet_ref):
    var = ex2_ref[...] - mu_ref[...] * mu_ref[...]
    a = gam_ref[...] / jnp.sqrt(var + BN_EPS)
    c = jnp.sum(bet_ref[...] - mu_ref[...] * a, axis=0, keepdims=True)
    dgn = h2 + LAMDA * (h2 * _dot(sft, a, ((1,), (0,))) + c)
    return dgn + h


def _apply_k(h_ref, h2_ref, sft_ref, mu_ref, ex2_ref, gam_ref, bet_ref,
             wd_ref, ws_ref, hn_ref, nd_ref, ns_ref):
    hn = _dgn_block(h_ref[...], h2_ref[...], sft_ref[...], mu_ref, ex2_ref,
                    gam_ref, bet_ref)
    hn_ref[...] = hn
    nd_ref[...] = _dot(hn, wd_ref[...], ((1,), (1,)))
    ns_ref[...] = _dot(hn, ws_ref[...], ((1,), (1,)))


def _apply(h, h2, sft, mu, ex2, gam, bet, wd, ws):
    small = lambda shape: pl.BlockSpec(shape, lambda i: (0, 0))
    return pl.pallas_call(
        _apply_k,
        grid=(N // BN,),
        in_specs=[
            pl.BlockSpec((BN, DIM1), lambda i: (i, 0)),
            pl.BlockSpec((BN, DIM1), lambda i: (i, 0)),
            pl.BlockSpec((BN, GROUPS), lambda i: (i, 0)),
            small((GROUPS, DIM1)),
            small((GROUPS, DIM1)),
            small((GROUPS, DIM1)),
            small((GROUPS, DIM1)),
            small((2 * DIM1, DIM1)),
            small((2 * DIM1, DIM1)),
        ],
        out_specs=[
            pl.BlockSpec((BN, DIM1), lambda i: (i, 0)),
            pl.BlockSpec((BN, 2 * DIM1), lambda i: (i, 0)),
            pl.BlockSpec((BN, 2 * DIM1), lambda i: (i, 0)),
        ],
        out_shape=[
            jax.ShapeDtypeStruct((N, DIM1), jnp.float32),
            jax.ShapeDtypeStruct((N, 2 * DIM1), jnp.float32),
            jax.ShapeDtypeStruct((N, 2 * DIM1), jnp.float32),
        ],
    )(h, h2, sft, mu, ex2, gam, bet, wd, ws)


def _pool_k(h_ref, h2_ref, sft_ref, mu_ref, ex2_ref, gam_ref, bet_ref, bt_ref,
            gsum_ref, cnts_ref):
    i = pl.program_id(0)
    hn = _dgn_block(h_ref[...], h2_ref[...], sft_ref[...], mu_ref, ex2_ref,
                    gam_ref, bet_ref)
    cols = lax.broadcasted_iota(jnp.int32, (BN, NUM_GRAPHS), 1)
    oneh = (bt_ref[...] == cols).astype(jnp.float32)
    gs = _dot(oneh, hn, ((0,), (0,)))
    cn = _dot(oneh, jnp.ones((BN, 1), jnp.float32), ((0,), (0,)))

    @pl.when(i == 0)
    def _():
        gsum_ref[...] = gs
        cnts_ref[...] = cn

    @pl.when(i > 0)
    def _():
        gsum_ref[...] += gs
        cnts_ref[...] += cn


def _pool(h, h2, sft, mu, ex2, gam, bet, bt2d):
    small = lambda shape: pl.BlockSpec(shape, lambda i: (0, 0))
    return pl.pallas_call(
        _pool_k,
        grid=(N // BN,),
        in_specs=[
            pl.BlockSpec((BN, DIM1), lambda i: (i, 0)),
            pl.BlockSpec((BN, DIM1), lambda i: (i, 0)),
            pl.BlockSpec((BN, GROUPS), lambda i: (i, 0)),
            small((GROUPS, DIM1)),
            small((GROUPS, DIM1)),
            small((GROUPS, DIM1)),
            small((GROUPS, DIM1)),
            pl.BlockSpec((BN, 1), lambda i: (i, 0)),
        ],
        out_specs=[
            small((NUM_GRAPHS, DIM1)),
            small((NUM_GRAPHS, 1)),
        ],
        out_shape=[
            jax.ShapeDtypeStruct((NUM_GRAPHS, DIM1), jnp.float32),
            jax.ShapeDtypeStruct((NUM_GRAPHS, 1), jnp.float32),
        ],
    )(h, h2, sft, mu, ex2, gam, bet, bt2d)


def _head_k(gsum_ref, cnts_ref,
            pwm_ref, pwr_ref, pbm_ref, pbr_ref,
            owm_ref, owr_ref, obm_ref, obr_ref,
            ew1_ref, eb1_ref, ew2_ref, eb2_ref,
            o_ref, kl_ref):
    g = gsum_ref[...] / jnp.maximum(cnts_ref[...], 1.0)

    def sp(v):
        return jnp.logaddexp(v, 0.0)

    sw1 = sp(pwr_ref[...])
    sb1 = sp(pbr_ref[...])
    w1 = pwm_ref[...] + sw1 * ew1_ref[...]
    b1 = pbm_ref[...] + sb1 * eb1_ref[...]
    h1 = jnp.maximum(_dot(g, w1, ((1,), (1,))) + b1, 0.0)

    sw2 = sp(owr_ref[...])
    sb2 = sp(obr_ref[...])
    w2 = owm_ref[...] + sw2 * ew2_ref[...]
    b2 = obm_ref[...] + sb2 * eb2_ref[...]
    o_ref[...] = jnp.sum(h1 * w2, axis=1, keepdims=True) + b2

    def kld(mu, sig):
        return 0.5 * jnp.sum(2.0 * (LOG_PRIOR - jnp.log(sig))
                             + (sig * sig + mu * mu) * INV_PR2 - 1.0)

    kl = (kld(pwm_ref[...], sw1) + kld(pbm_ref[...], sb1)
          + kld(owm_ref[...], sw2) + kld(obm_ref[...], sb2))
    kl_ref[...] = jnp.reshape(kl, (1, 1))


def _head(gsum, cnts, pwm, pwr, pbm, pbr, owm, owr, obm, obr,
          ew1, eb1, ew2, eb2):
    return pl.pallas_call(
        _head_k,
        out_shape=[
            jax.ShapeDtypeStruct((NUM_GRAPHS, 1), jnp.float32),
            jax.ShapeDtypeStruct((1, 1), jnp.float32),
        ],
    )(gsum, cnts, pwm, pwr, pbm, pbr, owm, owr, obm, obr,
      ew1, eb1, ew2, eb2)


def kernel(x, edge_index, edge_attr, batch, params):
    src = edge_index[0]
    dst = edge_index[1]
    convs = params['convs']

    wd = [jnp.concatenate([c['Wf'][:, 0:DIM1], c['Ws'][:, 0:DIM1]], axis=0)
          for c in convs]
    wsr = [jnp.concatenate([c['Wf'][:, DIM1:2 * DIM1], c['Ws'][:, DIM1:2 * DIM1]],
                           axis=0) for c in convs]
    we = [jnp.concatenate([c['Wf'][:, 2 * DIM1:], c['Ws'][:, 2 * DIM1:]], axis=0)
          for c in convs]
    be = [jnp.concatenate([c['bf'], c['bs']]).reshape(1, -1) for c in convs]

    h, nd, ns = _prep_nodes(x, params['pre_W'], params['pre_b'].reshape(1, -1),
                            wd[0], wsr[0])
    rs = [_edge_proj(edge_attr, we[l], be[l]) for l in range(GC_COUNT)]

    kb1, kb2 = jax.random.split(jax.random.key(42))
    k11, k12 = jax.random.split(kb1)
    k21, k22 = jax.random.split(kb2)
    ew1 = jax.random.normal(k11, (DIM2, DIM1), dtype=jnp.float32)
    eb1 = jax.random.normal(k12, (DIM2,), dtype=jnp.float32).reshape(1, -1)
    ew2 = jax.random.normal(k21, (1, DIM2), dtype=jnp.float32)
    eb2 = jax.random.normal(k22, (1,), dtype=jnp.float32).reshape(1, 1)

    for l in range(GC_COUNT):
        c = convs[l]
        s2 = _edge_pass(nd, ns, rs[l], dst, src)
        gb = c['gn_b'].reshape(1, -1)
        h2, sft, mu, ex2 = _moments(h, s2, c['gn_W'], gb)
        if l < GC_COUNT - 1:
            h, nd, ns = _apply(h, h2, sft, mu, ex2, c['gamma'], c['beta'],
                               wd[l + 1], wsr[l + 1])
        else:
            gsum, cnts = _pool(h, h2, sft, mu, ex2, c['gamma'], c['beta'],
                               batch.reshape(-1, 1))
            o, kl = _head(gsum, cnts,
                          params['post_Wmu'], params['post_Wrho'],
                          params['post_bmu'].reshape(1, -1),
                          params['post_brho'].reshape(1, -1),
                          params['out_Wmu'], params['out_Wrho'],
                          params['out_bmu'].reshape(1, 1),
                          params['out_brho'].reshape(1, 1),
                          ew1, eb1, ew2, eb2)

    return o.reshape(-1), kl.reshape(())
